# baseline (speedup 1.0000x reference)
import functools

import jax
import jax.numpy as jnp
from jax import lax
from jax.experimental import pallas as pl
from jax.experimental.pallas import tpu as pltpu
from jax.experimental.pallas import tpu_sc as plsc

MAX_ENTRIES = 16384
POS_BITS = 14
VALUE_DIM = 8
NUM_WRITES = 8192
NUM_READS = 4096

NC = 2
NS = 16
NW = NC * NS
POS_SHARD = MAX_ENTRIES // NS
READ_SHARD = NUM_READS // NW


def _sc_fused(wp_hbm, wv_hbm, rp_hbm, outT_hbm, stage_hbm,
              wp_v, wv_v, win_v, winf_v, rp_v, outT_v, cnt_v, need_v,
              sem_wv):
    cid = lax.axis_index("c")
    sid = lax.axis_index("s")
    wid = sid * NC + cid
    lo = sid * POS_SHARD

    with jax.named_scope("ph_copyin"):
        wv_cp = pltpu.async_copy(wv_hbm, wv_v, sem_wv)
        pltpu.sync_copy(wp_hbm, wp_v)

    lanes = jnp.arange(16, dtype=jnp.int32)
    neg1 = jnp.full((16,), -1, jnp.int32)

    def init_body(i, _):
        win_v[pl.ds(i * 16, 16)] = neg1
        return 0
    lax.fori_loop(0, POS_SHARD // 16, init_body, 0)

    def scan_body(t, lost_acc):
        v = wp_v[pl.ds(t * 16, 16)]
        m = (v >= lo) & (v < lo + POS_SHARD)
        lv = jnp.where(m, v - lo, 0)
        ivec = t * 16 + lanes
        plsc.store_scatter(win_v, [lv], ivec, mask=m)
        w2 = plsc.load_gather(win_v, [lv])
        return lost_acc | ((m & (w2 < ivec)).astype(jnp.int32))

    with jax.named_scope("ph_scan"):
        lost = lax.fori_loop(0, NUM_WRITES // 16, scan_body,
                             jnp.zeros((16,), jnp.int32))

    @pl.when(jnp.sum(lost) > 0)
    def _():
        def fix_body(t, _):
            v = wp_v[pl.ds(t * 16, 16)]
            m = (v >= lo) & (v < lo + POS_SHARD)
            lv = jnp.where(m, v - lo, 0)
            ivec = t * 16 + lanes

            def fcond(pend):
                return jnp.any(pend != 0)

            def fbody(pend):
                pm = pend != 0
                plsc.store_scatter(win_v, [lv], ivec, mask=pm)
                w2 = plsc.load_gather(win_v, [lv], mask=pm)
                return (pm & (w2 < ivec)).astype(jnp.int32)

            w0 = plsc.load_gather(win_v, [lv])
            lax.while_loop(fcond, fbody,
                           (m & (w0 < ivec)).astype(jnp.int32))
            return 0
        lax.fori_loop(0, NUM_WRITES // 16, fix_body, 0)

    with jax.named_scope("ph_exch"):
        pltpu.sync_copy(win_v, stage_hbm.at[cid, pl.ds(lo, POS_SHARD)])
        plsc.subcore_barrier()
        pltpu.sync_copy(stage_hbm.at[cid], winf_v)
        pltpu.sync_copy(rp_hbm.at[pl.ds(wid * READ_SHARD, READ_SHARD)], rp_v)
        wv_cp.wait()

    dsplat = [jnp.full((16,), d, jnp.int32) for d in range(VALUE_DIM)]

    for g in range(READ_SHARD // 16):
      with jax.named_scope(f"ph_read{g}"):
        sl = pl.ds(g * 16, 16)
        q = rp_v[sl]
        wq = plsc.load_gather(winf_v, [q])
        exact = wq >= 0
        wqc = jnp.maximum(wq, 0)
        cnt = exact.astype(jnp.int32)
        accs = [jnp.where(exact, plsc.load_gather(wv_v, [wqc, dsplat[d]]), 0.0)
                for d in range(VALUE_DIM)]
        nexact = jnp.logical_not(exact)
        for b in range(POS_BITS):
            cand = q ^ (1 << b)
            wb = plsc.load_gather(winf_v, [cand])
            sel = nexact & (wb >= 0)
            wbc = jnp.maximum(wb, 0)
            cnt = cnt + sel.astype(jnp.int32)
            for d in range(VALUE_DIM):
                accs[d] = accs[d] + jnp.where(
                    sel, plsc.load_gather(wv_v, [wbc, dsplat[d]]), 0.0)
        cnt_v[...] = cnt
        for d in range(VALUE_DIM):
            outT_v[d, sl] = accs[d]

        def ball_pass(n_iter, decode):
            need_v[...] = (cnt_v[...] == 0).astype(jnp.int32)

            @pl.when(jnp.sum(need_v[...]) > 0)
            def _():
                def body(j, _):
                    bits, ok = decode(j)

                    @pl.when(ok)
                    def _():
                        cand = q ^ bits
                        wb = plsc.load_gather(winf_v, [cand])
                        sel = (need_v[...] != 0) & (wb >= 0)
                        wbc = jnp.maximum(wb, 0)
                        cnt_v[...] = cnt_v[...] + sel.astype(jnp.int32)
                        for d in range(VALUE_DIM):
                            outT_v[d, sl] = outT_v[d, sl] + jnp.where(
                                sel, plsc.load_gather(wv_v, [wbc, dsplat[d]]),
                                0.0)
                    return 0
                lax.fori_loop(0, n_iter, body, 0)

        def decode2(j):
            b1, b2 = j // POS_BITS, j % POS_BITS
            return (1 << b1) + (1 << b2), b1 < b2

        def decode3(j):
            b1 = j // (POS_BITS * POS_BITS)
            r = j % (POS_BITS * POS_BITS)
            b2, b3 = r // POS_BITS, r % POS_BITS
            return (1 << b1) + (1 << b2) + (1 << b3), (b1 < b2) & (b2 < b3)

        ball_pass(POS_BITS * POS_BITS, decode2)
        ball_pass(POS_BITS * POS_BITS * POS_BITS, decode3)

        cntf = jnp.maximum(cnt_v[...], 1).astype(jnp.float32)
        for d in range(VALUE_DIM):
            outT_v[d, sl] = outT_v[d, sl] / cntf

    pltpu.sync_copy(outT_v, outT_hbm.at[wid])


_sc_fused_call = functools.partial(
    pl.kernel,
    out_type=[
        jax.ShapeDtypeStruct((NW, VALUE_DIM, READ_SHARD), jnp.float32),
        jax.ShapeDtypeStruct((NC, MAX_ENTRIES), jnp.int32),
    ],
    mesh=plsc.VectorSubcoreMesh(core_axis_name="c", subcore_axis_name="s"),
    compiler_params=pltpu.CompilerParams(use_tc_tiling_on_sc=False,
                                         needs_layout_passes=False),
    scratch_types=[
        pltpu.VMEM((NUM_WRITES,), jnp.int32),
        pltpu.VMEM((NUM_WRITES, VALUE_DIM), jnp.float32),
        pltpu.VMEM((POS_SHARD,), jnp.int32),
        pltpu.VMEM((MAX_ENTRIES,), jnp.int32),
        pltpu.VMEM((READ_SHARD,), jnp.int32),
        pltpu.VMEM((VALUE_DIM, READ_SHARD), jnp.float32),
        pltpu.VMEM((16,), jnp.int32),
        pltpu.VMEM((16,), jnp.int32),
        pltpu.SemaphoreType.DMA,
    ],
)(_sc_fused)


@jax.jit
def kernel(write_positions, write_values, read_positions):
    outT, _ = _sc_fused_call(write_positions, write_values, read_positions)
    return jnp.transpose(outT, (0, 2, 1)).reshape(NUM_READS, VALUE_DIM)


# speedup vs baseline: 1.0583x; 1.0583x over previous
import functools

import jax
import jax.numpy as jnp
from jax import lax
from jax.experimental import pallas as pl
from jax.experimental.pallas import tpu as pltpu
from jax.experimental.pallas import tpu_sc as plsc

MAX_ENTRIES = 16384
POS_BITS = 14
VALUE_DIM = 8
NUM_WRITES = 8192
NUM_READS = 4096

NC = 2
NS = 16
NW = NC * NS
POS_SHARD = MAX_ENTRIES // NS
READ_SHARD = NUM_READS // NW


def _sc_fused(wp_hbm, wv_hbm, rp_hbm, outT_hbm, stage_hbm,
              wp_v, wv_v, win_v, winf_v, rp_v, outT_v, cnt_v, need_v,
              sem_wv):
    cid = lax.axis_index("c")
    sid = lax.axis_index("s")
    wid = sid * NC + cid
    lo = sid * POS_SHARD

    with jax.named_scope("ph_copyin"):
        wv_cp = pltpu.async_copy(wv_hbm, wv_v, sem_wv)
        pltpu.sync_copy(wp_hbm, wp_v)

    lanes = jnp.arange(16, dtype=jnp.int32)
    neg1 = jnp.full((16,), -1, jnp.int32)

    def init_body(i, _):
        win_v[pl.ds(i * 16, 16)] = neg1
        return 0
    lax.fori_loop(0, POS_SHARD // 16, init_body, 0)

    def scan_body(t, lost_acc):
        v = wp_v[pl.ds(t * 16, 16)]
        m = (v >= lo) & (v < lo + POS_SHARD)
        lv = jnp.where(m, v - lo, 0)
        ivec = t * 16 + lanes
        plsc.store_scatter(win_v, [lv], ivec, mask=m)
        w2 = plsc.load_gather(win_v, [lv])
        return lost_acc | ((m & (w2 < ivec)).astype(jnp.int32))

    with jax.named_scope("ph_scan"):
        lost = lax.fori_loop(0, NUM_WRITES // 16, scan_body,
                             jnp.zeros((16,), jnp.int32))

    @pl.when(jnp.sum(lost) > 0)
    def _():
        def fix_body(t, _):
            v = wp_v[pl.ds(t * 16, 16)]
            m = (v >= lo) & (v < lo + POS_SHARD)
            lv = jnp.where(m, v - lo, 0)
            ivec = t * 16 + lanes

            def fcond(pend):
                return jnp.any(pend != 0)

            def fbody(pend):
                pm = pend != 0
                plsc.store_scatter(win_v, [lv], ivec, mask=pm)
                w2 = plsc.load_gather(win_v, [lv], mask=pm)
                return (pm & (w2 < ivec)).astype(jnp.int32)

            w0 = plsc.load_gather(win_v, [lv])
            lax.while_loop(fcond, fbody,
                           (m & (w0 < ivec)).astype(jnp.int32))
            return 0
        lax.fori_loop(0, NUM_WRITES // 16, fix_body, 0)

    with jax.named_scope("ph_exch"):
        pltpu.sync_copy(win_v, stage_hbm.at[cid, pl.ds(lo, POS_SHARD)])
        plsc.subcore_barrier()
        pltpu.sync_copy(stage_hbm.at[cid], winf_v)
        pltpu.sync_copy(rp_hbm.at[pl.ds(wid * READ_SHARD, READ_SHARD)], rp_v)
        wv_cp.wait()

    dsplat = [jnp.full((16,), d, jnp.int32) for d in range(VALUE_DIM)]

    def read_group(g, _):
        sl = pl.ds(g * 16, 16)
        q = rp_v[sl]
        wq = plsc.load_gather(winf_v, [q])
        exact = wq >= 0
        wqc = jnp.maximum(wq, 0)
        cnt = exact.astype(jnp.int32)
        accs = [jnp.where(exact, plsc.load_gather(wv_v, [wqc, dsplat[d]]), 0.0)
                for d in range(VALUE_DIM)]
        nexact = jnp.logical_not(exact)
        for b in range(POS_BITS):
            cand = q ^ (1 << b)
            wb = plsc.load_gather(winf_v, [cand])
            sel = nexact & (wb >= 0)
            wbc = jnp.maximum(wb, 0)
            cnt = cnt + sel.astype(jnp.int32)
            for d in range(VALUE_DIM):
                accs[d] = accs[d] + jnp.where(
                    sel, plsc.load_gather(wv_v, [wbc, dsplat[d]]), 0.0)
        cnt_v[...] = cnt
        for d in range(VALUE_DIM):
            outT_v[d, sl] = accs[d]

        def ball_pass(n_iter, decode):
            need_v[...] = (cnt_v[...] == 0).astype(jnp.int32)

            @pl.when(jnp.sum(need_v[...]) > 0)
            def _():
                def body(j, _):
                    bits, ok = decode(j)

                    @pl.when(ok)
                    def _():
                        cand = q ^ bits
                        wb = plsc.load_gather(winf_v, [cand])
                        sel = (need_v[...] != 0) & (wb >= 0)
                        wbc = jnp.maximum(wb, 0)
                        cnt_v[...] = cnt_v[...] + sel.astype(jnp.int32)
                        for d in range(VALUE_DIM):
                            outT_v[d, sl] = outT_v[d, sl] + jnp.where(
                                sel, plsc.load_gather(wv_v, [wbc, dsplat[d]]),
                                0.0)
                    return 0
                lax.fori_loop(0, n_iter, body, 0)

        def decode2(j):
            b1, b2 = j // POS_BITS, j % POS_BITS
            return (1 << b1) + (1 << b2), b1 < b2

        def decode3(j):
            b1 = j // (POS_BITS * POS_BITS)
            r = j % (POS_BITS * POS_BITS)
            b2, b3 = r // POS_BITS, r % POS_BITS
            return (1 << b1) + (1 << b2) + (1 << b3), (b1 < b2) & (b2 < b3)

        ball_pass(POS_BITS * POS_BITS, decode2)
        ball_pass(POS_BITS * POS_BITS * POS_BITS, decode3)

        cntf = jnp.maximum(cnt_v[...], 1).astype(jnp.float32)
        for d in range(VALUE_DIM):
            outT_v[d, sl] = outT_v[d, sl] / cntf
        return 0

    with jax.named_scope("ph_read"):
        lax.fori_loop(0, READ_SHARD // 16, read_group, 0)

    pltpu.sync_copy(outT_v, outT_hbm.at[wid])


_sc_fused_call = functools.partial(
    pl.kernel,
    out_type=[
        jax.ShapeDtypeStruct((NW, VALUE_DIM, READ_SHARD), jnp.float32),
        jax.ShapeDtypeStruct((NC, MAX_ENTRIES), jnp.int32),
    ],
    mesh=plsc.VectorSubcoreMesh(core_axis_name="c", subcore_axis_name="s"),
    compiler_params=pltpu.CompilerParams(use_tc_tiling_on_sc=False,
                                         needs_layout_passes=False),
    scratch_types=[
        pltpu.VMEM((NUM_WRITES,), jnp.int32),
        pltpu.VMEM((NUM_WRITES, VALUE_DIM), jnp.float32),
        pltpu.VMEM((POS_SHARD,), jnp.int32),
        pltpu.VMEM((MAX_ENTRIES,), jnp.int32),
        pltpu.VMEM((READ_SHARD,), jnp.int32),
        pltpu.VMEM((VALUE_DIM, READ_SHARD), jnp.float32),
        pltpu.VMEM((16,), jnp.int32),
        pltpu.VMEM((16,), jnp.int32),
        pltpu.SemaphoreType.DMA,
    ],
)(_sc_fused)


@jax.jit
def kernel(write_positions, write_values, read_positions):
    outT, _ = _sc_fused_call(write_positions, write_values, read_positions)
    return jnp.transpose(outT, (0, 2, 1)).reshape(NUM_READS, VALUE_DIM)
